# baseline (speedup 1.0000x reference)
import functools

import jax
import jax.numpy as jnp
from jax import lax
from jax.experimental import pallas as pl
from jax.experimental.pallas import tpu as pltpu
from jax.experimental.pallas import tpu_sc as plsc

_SU, _SL, _TU, _TL = 1000.0, 0.0, 500.0, 0.0
_TM = 50
_DPAD = 128
_ROWS_PER_WORKER = 128


def _sc_gather(table, idx, n_rows):
    info = plsc.get_sparse_core_info()
    b_per_w = _ROWS_PER_WORKER
    n_active = n_rows // b_per_w
    d = table.shape[1]
    mesh = plsc.VectorSubcoreMesh(core_axis_name="c", subcore_axis_name="s")

    @functools.partial(
        pl.kernel, mesh=mesh,
        out_type=jax.ShapeDtypeStruct((n_rows, d), jnp.float32),
        scratch_types=[
            pltpu.VMEM((b_per_w,), jnp.int32),
            pltpu.VMEM((b_per_w, d), jnp.float32),
            pltpu.SemaphoreType.DMA,
        ],
    )
    def k(table_hbm, idx_hbm, out_hbm, idx_v, rows_v, sem):
        wid = lax.axis_index("s") * info.num_cores + lax.axis_index("c")

        @pl.when(wid < n_active)
        def _():
            base = wid * b_per_w
            pltpu.sync_copy(idx_hbm.at[pl.ds(base, b_per_w)], idx_v)
            pltpu.async_copy(table_hbm.at[idx_v], rows_v, sem).wait()
            pltpu.sync_copy(rows_v, out_hbm.at[pl.ds(base, b_per_w)])

    return k(table, idx)


def _expand_kernel(l, len_ref, lmax_ref,
                   ds_ref, su_ref, sl_ref, tu_ref, tl_ref,
                   out_ref):
    b = pl.program_id(0)
    j = pl.program_id(1)
    _, tm, lq, lanes = out_ref.shape
    m0 = j * tm

    tlen = len_ref[b]
    lmax = lmax_ref[0]
    v2 = (jax.lax.broadcasted_iota(jnp.int32, (tm, 1), 0) + m0) < tlen
    col_ok = jax.lax.broadcasted_iota(jnp.int32, (tm, l), 1) < lmax
    ds = jnp.where(v2 & col_ok, ds_ref[0, :, :l], 0.0)

    esl = jnp.where(v2, sl_ref[1:2, :], sl_ref[0:1, :])
    esu = jnp.where(v2, su_ref[1:2, :], su_ref[0:1, :])
    etl = jnp.where(v2, tl_ref[1:2, :], tl_ref[0:1, :])
    etu = jnp.where(v2, tu_ref[1:2, :], tu_ref[0:1, :])
    a_v = (esl * _SU - esu * _SL) * (1.0 / (_SU - _SL)) + \
          (etl * _TU - etu * _TL) * (1.0 / (_TU - _TL))
    b_v = (esu - esl) * (1.0 / (_SU - _SL)) + \
          (etu - etl) * (1.0 / (_TU - _TL))

    a2 = jnp.concatenate([a_v, a_v], axis=1)
    b2 = jnp.concatenate([b_v, b_v], axis=1)
    e = a_v.shape[1]
    p = jnp.repeat(ds.reshape(tm, lq, lanes // e), e, axis=2)
    out_ref[0] = a2[:, None, :] + b2[:, None, :] * p


def kernel(traj_loc, mat2, vec, traj_len, l_max, emb_su, emb_sl, emb_tu, emb_tl):
    del vec
    b_sz, m_sz = traj_loc.shape
    n_loc, l_sz = mat2.shape
    e_sz = emb_su.shape[1]
    tm = _TM if m_sz % _TM == 0 else m_sz
    steps_per_b = m_sz // tm
    grid = (b_sz, steps_per_b)
    lq = l_sz * e_sz // 128

    table = jnp.pad(mat2, ((1, 0), (0, _DPAD - l_sz)))
    n_pairs = b_sz * m_sz
    idx = traj_loc.astype(jnp.int32).reshape(-1)
    ds_rows = _sc_gather(table, idx, n_pairs)
    ds3 = ds_rows.reshape(n_pairs // tm, tm, _DPAD)

    lmax_arr = jnp.asarray(l_max, jnp.int32).reshape(1)
    full = lambda bb, jj, *refs: (0, 0)

    out = pl.pallas_call(
        functools.partial(_expand_kernel, l_sz),
        grid_spec=pltpu.PrefetchScalarGridSpec(
            num_scalar_prefetch=2,
            grid=grid,
            in_specs=[
                pl.BlockSpec((1, tm, _DPAD),
                             lambda bb, jj, *refs: (bb * steps_per_b + jj, 0, 0)),
                pl.BlockSpec((2, e_sz), full),
                pl.BlockSpec((2, e_sz), full),
                pl.BlockSpec((2, e_sz), full),
                pl.BlockSpec((2, e_sz), full),
            ],
            out_specs=pl.BlockSpec((1, tm, lq, 128),
                                   lambda bb, jj, *refs: (bb, jj, 0, 0)),
        ),
        out_shape=jax.ShapeDtypeStruct((b_sz, m_sz, lq, 128), jnp.float32),
    )(traj_len.astype(jnp.int32), lmax_arr,
      ds3, emb_su, emb_sl, emb_tu, emb_tl)
    return out.reshape(b_sz, m_sz, l_sz, e_sz)


# speedup vs baseline: 1.8413x; 1.8413x over previous
import functools

import jax
import jax.numpy as jnp
from jax import lax
from jax.experimental import pallas as pl
from jax.experimental.pallas import tpu as pltpu
from jax.experimental.pallas import tpu_sc as plsc

_SU, _SL, _TU, _TL = 1000.0, 0.0, 500.0, 0.0
_TM = 100
_DPAD = 128
_ROWS_PER_WORKER = 128


def _sc_gather(table, idx, n_rows):
    info = plsc.get_sparse_core_info()
    b_per_w = _ROWS_PER_WORKER
    n_active = n_rows // b_per_w
    d = table.shape[1]
    mesh = plsc.VectorSubcoreMesh(core_axis_name="c", subcore_axis_name="s")

    @functools.partial(
        pl.kernel, mesh=mesh,
        out_type=jax.ShapeDtypeStruct((n_rows, d), jnp.float32),
        scratch_types=[
            pltpu.VMEM((b_per_w,), jnp.int32),
            pltpu.VMEM((b_per_w, d), jnp.float32),
            pltpu.SemaphoreType.DMA,
        ],
    )
    def k(table_hbm, idx_hbm, out_hbm, idx_v, rows_v, sem):
        wid = lax.axis_index("s") * info.num_cores + lax.axis_index("c")

        @pl.when(wid < n_active)
        def _():
            base = wid * b_per_w
            pltpu.sync_copy(idx_hbm.at[pl.ds(base, b_per_w)], idx_v)
            pltpu.async_copy(table_hbm.at[idx_v], rows_v, sem).wait()
            pltpu.sync_copy(rows_v, out_hbm.at[pl.ds(base, b_per_w)])

    return k(table, idx)


def _expand_kernel(l, len_ref, lmax_ref,
                   ds_ref, msel_ref, trep_ref,
                   su_ref, sl_ref, tu_ref, tl_ref,
                   out_ref):
    b = pl.program_id(0)
    j = pl.program_id(1)
    _, tm, lanes = out_ref.shape
    m0 = j * tm

    tlen = len_ref[b]
    lmax = lmax_ref[0]
    v2 = (jax.lax.broadcasted_iota(jnp.int32, (tm, 1), 0) + m0) < tlen
    col_ok = jax.lax.broadcasted_iota(jnp.int32, (tm, l), 1) < lmax
    ds = jnp.where(v2 & col_ok, ds_ref[0, :, :l], 0.0)

    a_tab = (sl_ref[...] * _SU - su_ref[...] * _SL) * (1.0 / (_SU - _SL)) + \
            (tl_ref[...] * _TU - tu_ref[...] * _TL) * (1.0 / (_TU - _TL))
    b_tab = (su_ref[...] - sl_ref[...]) * (1.0 / (_SU - _SL)) + \
            (tu_ref[...] - tl_ref[...]) * (1.0 / (_TU - _TL))
    dims = (((1,), (0,)), ((), ()))
    a_rep = lax.dot_general(a_tab, trep_ref[...], dims,
                            preferred_element_type=jnp.float32)
    b_rep = lax.dot_general(b_tab, trep_ref[...], dims,
                            preferred_element_type=jnp.float32)
    a_row = jnp.where(v2, a_rep[1:2, :], a_rep[0:1, :])
    b_row = jnp.where(v2, b_rep[1:2, :], b_rep[0:1, :])

    p2 = lax.dot_general(ds, msel_ref[...], dims,
                         preferred_element_type=jnp.float32)
    out_ref[0] = a_row + b_row * p2


def kernel(traj_loc, mat2, vec, traj_len, l_max, emb_su, emb_sl, emb_tu, emb_tl):
    del vec
    b_sz, m_sz = traj_loc.shape
    n_loc, l_sz = mat2.shape
    e_sz = emb_su.shape[1]
    tm = _TM if m_sz % _TM == 0 else m_sz
    steps_per_b = m_sz // tm
    grid = (b_sz, steps_per_b)
    le = l_sz * e_sz

    table = jnp.pad(mat2, ((1, 0), (0, _DPAD - l_sz)))
    n_pairs = b_sz * m_sz
    idx = traj_loc.astype(jnp.int32).reshape(-1)
    ds_rows = _sc_gather(table, idx, n_pairs)
    ds3 = ds_rows.reshape(n_pairs // tm, tm, _DPAD)

    k_ar = jnp.arange(le, dtype=jnp.int32)
    msel = (k_ar[None, :] // e_sz == jnp.arange(l_sz, dtype=jnp.int32)[:, None]
            ).astype(jnp.float32)
    trep = (k_ar[None, :] % e_sz == jnp.arange(e_sz, dtype=jnp.int32)[:, None]
            ).astype(jnp.float32)

    lmax_arr = jnp.asarray(l_max, jnp.int32).reshape(1)
    full = lambda bb, jj, *refs: (0, 0)

    out = pl.pallas_call(
        functools.partial(_expand_kernel, l_sz),
        grid_spec=pltpu.PrefetchScalarGridSpec(
            num_scalar_prefetch=2,
            grid=grid,
            in_specs=[
                pl.BlockSpec((1, tm, _DPAD),
                             lambda bb, jj, *refs: (bb * steps_per_b + jj, 0, 0)),
                pl.BlockSpec((l_sz, le), full),
                pl.BlockSpec((e_sz, le), full),
                pl.BlockSpec((2, e_sz), full),
                pl.BlockSpec((2, e_sz), full),
                pl.BlockSpec((2, e_sz), full),
                pl.BlockSpec((2, e_sz), full),
            ],
            out_specs=pl.BlockSpec((1, tm, le),
                                   lambda bb, jj, *refs: (bb, jj, 0)),
        ),
        out_shape=jax.ShapeDtypeStruct((b_sz, m_sz, le), jnp.float32),
    )(traj_len.astype(jnp.int32), lmax_arr,
      ds3, msel, trep, emb_su, emb_sl, emb_tu, emb_tl)
    return out.reshape(b_sz, m_sz, l_sz, e_sz)


# speedup vs baseline: 2.2369x; 1.2148x over previous
import functools

import jax
import jax.numpy as jnp
from jax import lax
from jax.experimental import pallas as pl
from jax.experimental.pallas import tpu as pltpu
from jax.experimental.pallas import tpu_sc as plsc

_SU, _SL, _TU, _TL = 1000.0, 0.0, 500.0, 0.0
_BB = 2
_DPAD = 128
_ROWS_PER_WORKER = 128


def _sc_gather(table, idx, n_rows):
    info = plsc.get_sparse_core_info()
    b_per_w = _ROWS_PER_WORKER
    n_active = n_rows // b_per_w
    d = table.shape[1]
    mesh = plsc.VectorSubcoreMesh(core_axis_name="c", subcore_axis_name="s")

    @functools.partial(
        pl.kernel, mesh=mesh,
        out_type=jax.ShapeDtypeStruct((n_rows, d), jnp.float32),
        scratch_types=[
            pltpu.VMEM((b_per_w,), jnp.int32),
            pltpu.VMEM((b_per_w, d), jnp.float32),
            pltpu.SemaphoreType.DMA,
        ],
    )
    def k(table_hbm, idx_hbm, out_hbm, idx_v, rows_v, sem):
        wid = lax.axis_index("s") * info.num_cores + lax.axis_index("c")

        @pl.when(wid < n_active)
        def _():
            base = wid * b_per_w
            pltpu.sync_copy(idx_hbm.at[pl.ds(base, b_per_w)], idx_v)
            pltpu.async_copy(table_hbm.at[idx_v], rows_v, sem).wait()
            pltpu.sync_copy(rows_v, out_hbm.at[pl.ds(base, b_per_w)])

    return k(table, idx)


def _expand_kernel(lmax_ref,
                   ds_ref, tlen_ref, su_ref, sl_ref, tu_ref, tl_ref,
                   out_ref):
    bb, m_sz, l, e = out_ref.shape
    rows = bb * m_sz

    lmax = lmax_ref[0]
    m_pp = jax.lax.broadcasted_iota(jnp.int32, (rows, 1), 0) % m_sz
    v2 = m_pp < tlen_ref[0]
    col_ok = jax.lax.broadcasted_iota(jnp.int32, (rows, l), 1) < lmax
    ds = jnp.where(v2 & col_ok, ds_ref[0, :, :l], 0.0)

    a_tab = (sl_ref[...] * _SU - su_ref[...] * _SL) * (1.0 / (_SU - _SL)) + \
            (tl_ref[...] * _TU - tu_ref[...] * _TL) * (1.0 / (_TU - _TL))
    b_tab = (su_ref[...] - sl_ref[...]) * (1.0 / (_SU - _SL)) + \
            (tu_ref[...] - tl_ref[...]) * (1.0 / (_TU - _TL))
    a_v = jnp.where(v2, a_tab[1:2, :], a_tab[0:1, :])
    b_v = jnp.where(v2, b_tab[1:2, :], b_tab[0:1, :])

    val = a_v[:, None, :] + b_v[:, None, :] * ds[:, :, None]
    for t in range(bb):
        out_ref[t] = val[t * m_sz:(t + 1) * m_sz]


def kernel(traj_loc, mat2, vec, traj_len, l_max, emb_su, emb_sl, emb_tu, emb_tl):
    del vec
    b_sz, m_sz = traj_loc.shape
    n_loc, l_sz = mat2.shape
    e_sz = emb_su.shape[1]
    bb = _BB if b_sz % _BB == 0 else 1
    grid = (b_sz // bb,)
    rows = bb * m_sz

    table = jnp.pad(mat2, ((1, 0), (0, _DPAD - l_sz)))
    n_pairs = b_sz * m_sz
    idx = traj_loc.astype(jnp.int32).reshape(-1)
    ds_rows = _sc_gather(table, idx, n_pairs)
    ds3 = ds_rows.reshape(n_pairs // rows, rows, _DPAD)

    tlen_pp = jnp.repeat(traj_len.astype(jnp.int32), m_sz
                         ).reshape(n_pairs // rows, rows, 1)

    lmax_arr = jnp.asarray(l_max, jnp.int32).reshape(1)
    full = lambda s, *refs: (0, 0)

    out = pl.pallas_call(
        _expand_kernel,
        grid_spec=pltpu.PrefetchScalarGridSpec(
            num_scalar_prefetch=1,
            grid=grid,
            in_specs=[
                pl.BlockSpec((1, rows, _DPAD), lambda s, *refs: (s, 0, 0)),
                pl.BlockSpec((1, rows, 1), lambda s, *refs: (s, 0, 0)),
                pl.BlockSpec((2, e_sz), full),
                pl.BlockSpec((2, e_sz), full),
                pl.BlockSpec((2, e_sz), full),
                pl.BlockSpec((2, e_sz), full),
            ],
            out_specs=pl.BlockSpec((bb, m_sz, l_sz, e_sz),
                                   lambda s, *refs: (s, 0, 0, 0)),
        ),
        out_shape=jax.ShapeDtypeStruct((b_sz, m_sz, l_sz, e_sz), jnp.float32),
    )(lmax_arr, ds3, tlen_pp, emb_su, emb_sl, emb_tu, emb_tl)
    return out
